# baseline (speedup 1.0000x reference)
import functools

import jax
import jax.numpy as jnp
from jax import lax
from jax.experimental import pallas as pl
from jax.experimental.pallas import tpu as pltpu
from jax.experimental.pallas import tpu_sc as plsc

NC = 2
NS = 16
NW = NC * NS
WIN = 128
PAD_ROWS = 112
_HIGHEST = jax.lax.Precision.HIGHEST


def _flat_wid():
    return lax.axis_index("c") * NS + lax.axis_index("s")


def _deg_partials(dst3, ones_rows, zeros_d, n_acc, width):
    k_win = dst3.shape[1]
    rows_sub = n_acc // NS
    mesh = plsc.VectorSubcoreMesh(core_axis_name="c", subcore_axis_name="s")

    @functools.partial(
        pl.kernel,
        mesh=mesh,
        out_type=jax.ShapeDtypeStruct((NC, n_acc, width), jnp.float32),
        scratch_types=[
            pltpu.VMEM((k_win, WIN), jnp.int32),
            pltpu.VMEM((WIN, width), jnp.float32),
            pltpu.VMEM_SHARED((n_acc, width), jnp.float32),
            pltpu.SemaphoreType.DMA,
        ],
    )
    def deg_kernel(dst_hbm, ones_hbm, zeros_hbm, out_hbm, dst_v, ones_v,
                   acc_sh, sem):
        c = lax.axis_index("c")
        s = lax.axis_index("s")
        wid = _flat_wid()
        pltpu.sync_copy(zeros_hbm.at[pl.ds(s * rows_sub, rows_sub)],
                        acc_sh.at[pl.ds(s * rows_sub, rows_sub)])
        pltpu.sync_copy(dst_hbm.at[wid], dst_v)
        pltpu.sync_copy(ones_hbm, ones_v)
        plsc.subcore_barrier()

        @pl.loop(0, k_win)
        def _(j):
            pltpu.async_copy(ones_v, acc_sh.at[dst_v.at[j]], sem, add=True)

        @pl.loop(0, k_win)
        def _(j):
            pltpu.make_async_copy(ones_hbm, ones_v, sem).wait()

        plsc.subcore_barrier()
        pltpu.sync_copy(acc_sh.at[pl.ds(s * rows_sub, rows_sub)],
                        out_hbm.at[c, pl.ds(s * rows_sub, rows_sub)])

    return deg_kernel(dst3, ones_rows, zeros_d)


def _agg_partials(src3, dst3, p, zeros_d, n, n_acc):
    del n
    k_win = src3.shape[1]
    d_feat = p.shape[1]
    rows_sub = n_acc // NS
    idxc = 16
    n_chunks = k_win // idxc
    mesh = plsc.VectorSubcoreMesh(core_axis_name="c", subcore_axis_name="s")

    @functools.partial(
        pl.kernel,
        mesh=mesh,
        out_type=jax.ShapeDtypeStruct((NC, n_acc, d_feat), jnp.float32),
        scratch_types=[
            pltpu.VMEM((idxc, WIN), jnp.int32),
            pltpu.VMEM((idxc, WIN), jnp.int32),
            pltpu.VMEM((WIN, d_feat), jnp.float32),
            pltpu.VMEM((WIN, d_feat), jnp.float32),
            pltpu.VMEM_SHARED((n_acc, d_feat), jnp.float32),
            pltpu.SemaphoreType.DMA,
            pltpu.SemaphoreType.DMA,
        ],
    )
    def agg_kernel(src_hbm, dst_hbm, p_hbm, zeros_hbm, out_hbm,
                   src_v, dst_v, rows0_v, rows1_v, acc_sh, sem0, sem1):
        c = lax.axis_index("c")
        s = lax.axis_index("s")
        wid = _flat_wid()
        pltpu.sync_copy(zeros_hbm.at[pl.ds(s * rows_sub, rows_sub)],
                        acc_sh.at[pl.ds(s * rows_sub, rows_sub)])
        plsc.subcore_barrier()

        @pl.loop(0, n_chunks)
        def _(cb):
            pltpu.sync_copy(src_hbm.at[wid, pl.ds(cb * idxc, idxc)], src_v)
            pltpu.sync_copy(dst_hbm.at[wid, pl.ds(cb * idxc, idxc)], dst_v)
            pltpu.async_copy(p_hbm.at[src_v.at[0]], rows0_v, sem0)

            @pl.loop(0, idxc // 2)
            def _(t):
                j0 = 2 * t
                pltpu.async_copy(p_hbm.at[src_v.at[j0 + 1]], rows1_v, sem1)
                pltpu.make_async_copy(p_hbm.at[src_v.at[j0]], rows0_v,
                                      sem0).wait()
                pltpu.sync_copy(rows0_v, acc_sh.at[dst_v.at[j0]], add=True)

                @pl.when(t < idxc // 2 - 1)
                def _():
                    pltpu.async_copy(p_hbm.at[src_v.at[j0 + 2]], rows0_v, sem0)

                pltpu.make_async_copy(p_hbm.at[src_v.at[j0 + 1]], rows1_v,
                                      sem1).wait()
                pltpu.sync_copy(rows1_v, acc_sh.at[dst_v.at[j0 + 1]], add=True)

        plsc.subcore_barrier()
        pltpu.sync_copy(acc_sh.at[pl.ds(s * rows_sub, rows_sub)],
                        out_hbm.at[c, pl.ds(s * rows_sub, rows_sub)])

    return agg_kernel(src3, dst3, p, zeros_d)


def _dinv_from_parts(degp, n):
    deg = degp[0, :n, 0:1] + degp[1, :n, 0:1] + 1.0
    return 1.0 / jnp.sqrt(deg)


def _mm_body(x_ref, w_ref, o_ref):
    o_ref[...] = jnp.dot(x_ref[...], w_ref[...],
                         preferred_element_type=jnp.float32,
                         precision=_HIGHEST)


def _scale_body(h_ref, degp_ref, p_ref):
    n = h_ref.shape[0]
    p_ref[...] = h_ref[...] * _dinv_from_parts(degp_ref[...], n)


def _hidden_body(a_ref, p1_ref, degp_ref, b1_ref, p2_ref):
    n = p1_ref.shape[0]
    dinv = _dinv_from_parts(degp_ref[...], n)
    a = a_ref[...]
    pre = (a[0, :n] + a[1, :n] + p1_ref[...]) * dinv + b1_ref[...]
    p2_ref[...] = jnp.maximum(pre, 0.0) * dinv


def _final_body(a_ref, p2_ref, degp_ref, wmu_ref, bmu_ref, wlv_ref, blv_ref,
                mu_ref, lv_ref):
    n = p2_ref.shape[0]
    dinv = _dinv_from_parts(degp_ref[...], n)
    a = a_ref[...]
    z = (a[0, :n] + a[1, :n] + p2_ref[...]) * dinv
    mu_ref[...] = jnp.dot(z, wmu_ref[...], preferred_element_type=jnp.float32,
                          precision=_HIGHEST) + bmu_ref[...]
    lv_ref[...] = jnp.dot(z, wlv_ref[...], preferred_element_type=jnp.float32,
                          precision=_HIGHEST) + blv_ref[...]


def _f32(*shape):
    return jax.ShapeDtypeStruct(shape, jnp.float32)


def kernel(x, edge_index, W1, b1, W_mu, b_mu, W_lv, b_lv):
    n, d_in = x.shape
    h1_dim = W1.shape[1]
    h2_dim = W_mu.shape[1]
    e = edge_index.shape[1]

    chunk = NW * WIN * 2
    e_pad = -(-e // chunk) * chunk
    pad = e_pad - e
    k_win = e_pad // (NW * WIN)
    n_acc = n + PAD_ROWS

    src = edge_index[0]
    dst = edge_index[1]
    pad_src = jnp.zeros((pad,), jnp.int32)
    pad_dst = n + (jnp.arange(pad, dtype=jnp.int32) % PAD_ROWS)
    src3 = jnp.concatenate([src, pad_src]).reshape(k_win * WIN, NW).T \
        .reshape(NW, k_win, WIN)
    dst3 = jnp.concatenate([dst, pad_dst]).reshape(k_win * WIN, NW).T \
        .reshape(NW, k_win, WIN)

    ones_rows = jnp.ones((WIN, h1_dim), jnp.float32)
    zeros_d = jnp.zeros((n_acc, h1_dim), jnp.float32)

    degp = _deg_partials(dst3, ones_rows, zeros_d, n_acc, h1_dim)
    h1 = pl.pallas_call(_mm_body, out_shape=_f32(n, h1_dim))(x, W1)

    p1 = pl.pallas_call(_scale_body, out_shape=_f32(n, h1_dim))(h1, degp)

    a1 = _agg_partials(src3, dst3, p1, zeros_d, n, n_acc)

    p2 = pl.pallas_call(_hidden_body, out_shape=_f32(n, h1_dim))(
        a1, p1, degp, b1.reshape(1, h1_dim))

    a2 = _agg_partials(src3, dst3, p2, zeros_d, n, n_acc)

    mu, lv = pl.pallas_call(
        _final_body, out_shape=(_f32(n, h2_dim), _f32(n, h2_dim)))(
        a2, p2, degp, W_mu, b_mu.reshape(1, h2_dim), W_lv,
        b_lv.reshape(1, h2_dim))
    return (mu, lv)


# speedup vs baseline: 1.4548x; 1.4548x over previous
import functools

import jax
import jax.numpy as jnp
from jax import lax
from jax.experimental import pallas as pl
from jax.experimental.pallas import tpu as pltpu
from jax.experimental.pallas import tpu_sc as plsc

NC = 2
NS = 16
NW = NC * NS
WIN = 128
IDXC = 16
PAD_ROWS = 112
_HIGHEST = jax.lax.Precision.HIGHEST
_UNTILED = pltpu.CompilerParams(use_tc_tiling_on_sc=False)

_MESH = plsc.VectorSubcoreMesh(core_axis_name="c", subcore_axis_name="s")


def _flat_wid():
    return lax.axis_index("c") * NS + lax.axis_index("s")


def _deg_partials(dst3, ones_rows, zeros16, n_acc):
    k_win = dst3.shape[1]
    rows_sub = n_acc // NS

    @functools.partial(
        pl.kernel,
        mesh=_MESH,
        out_type=jax.ShapeDtypeStruct((NC, n_acc, 16), jnp.float32),
        scratch_types=[
            pltpu.VMEM((k_win, WIN), jnp.int32),
            pltpu.VMEM((WIN, 16), jnp.float32),
            pltpu.VMEM_SHARED((n_acc, 16), jnp.float32),
            pltpu.SemaphoreType.DMA,
        ],
        compiler_params=_UNTILED,
    )
    def deg_kernel(dst_hbm, ones_hbm, zeros_hbm, out_hbm, dst_v, ones_v,
                   acc_sh, sem):
        c = lax.axis_index("c")
        s = lax.axis_index("s")
        wid = _flat_wid()
        pltpu.sync_copy(zeros_hbm.at[pl.ds(s * rows_sub, rows_sub)],
                        acc_sh.at[pl.ds(s * rows_sub, rows_sub)])
        pltpu.sync_copy(dst_hbm.at[wid], dst_v)
        pltpu.sync_copy(ones_hbm, ones_v)
        plsc.subcore_barrier()

        @pl.loop(0, k_win)
        def _(j):
            pltpu.async_copy(ones_v, acc_sh.at[dst_v.at[j]], sem, add=True)

        @pl.loop(0, k_win)
        def _(j):
            pltpu.make_async_copy(ones_hbm, ones_v, sem).wait()

        plsc.subcore_barrier()
        pltpu.sync_copy(acc_sh.at[pl.ds(s * rows_sub, rows_sub)],
                        out_hbm.at[c, pl.ds(s * rows_sub, rows_sub)])

    return deg_kernel(dst3, ones_rows, zeros16)


def _agg_partials(src3, dst3, ph, zeros_d, n_acc):
    k_win = src3.shape[1]
    n_tab = ph.shape[1]
    dh = ph.shape[2]
    rows_sub = n_acc // NS
    tab_sub = n_tab // NS
    n_chunks = k_win // IDXC

    @functools.partial(
        pl.kernel,
        mesh=_MESH,
        out_type=jax.ShapeDtypeStruct((2, NC, n_acc, dh), jnp.float32),
        scratch_types=[
            pltpu.VMEM((IDXC, WIN), jnp.int32),
            pltpu.VMEM((IDXC, WIN), jnp.int32),
            pltpu.VMEM((WIN, dh), jnp.float32),
            pltpu.VMEM_SHARED((n_tab, dh), jnp.float32),
            pltpu.VMEM_SHARED((n_acc, dh), jnp.float32),
            pltpu.SemaphoreType.DMA,
        ],
        compiler_params=_UNTILED,
    )
    def agg_kernel(src_hbm, dst_hbm, ph_hbm, zeros_hbm, out_hbm,
                   src_v, dst_v, rows_v, tab_sh, acc_sh, sem):
        c = lax.axis_index("c")
        s = lax.axis_index("s")
        wid = _flat_wid()

        @pl.loop(0, 2)
        def _(h):
            pltpu.sync_copy(ph_hbm.at[h, pl.ds(s * tab_sub, tab_sub)],
                            tab_sh.at[pl.ds(s * tab_sub, tab_sub)])
            pltpu.sync_copy(zeros_hbm.at[pl.ds(s * rows_sub, rows_sub)],
                            acc_sh.at[pl.ds(s * rows_sub, rows_sub)])
            plsc.subcore_barrier()

            @pl.loop(0, n_chunks)
            def _(cb):
                pltpu.sync_copy(src_hbm.at[wid, pl.ds(cb * IDXC, IDXC)], src_v)
                pltpu.sync_copy(dst_hbm.at[wid, pl.ds(cb * IDXC, IDXC)], dst_v)

                @pl.loop(0, IDXC)
                def _(j):
                    pltpu.async_copy(tab_sh.at[src_v.at[j]], rows_v,
                                     sem).wait()
                    pltpu.sync_copy(rows_v, acc_sh.at[dst_v.at[j]], add=True)

            plsc.subcore_barrier()
            pltpu.sync_copy(acc_sh.at[pl.ds(s * rows_sub, rows_sub)],
                            out_hbm.at[h, c, pl.ds(s * rows_sub, rows_sub)])

    return agg_kernel(src3, dst3, ph, zeros_d)


BLK = 400


def _dinv_from_parts(degp):
    deg = degp[0, :, 0:1] + degp[1, :, 0:1] + 1.0
    return 1.0 / jnp.sqrt(deg)


def _halves(v, dh):
    return jnp.stack([v[:, :dh], v[:, dh:]], axis=0)


def _mm_body(x_ref, w_ref, o_ref):
    o_ref[...] = jnp.dot(x_ref[...], w_ref[...],
                         preferred_element_type=jnp.float32,
                         precision=_HIGHEST)


def _scale_body(h_ref, degp_ref, p_ref):
    dh = p_ref.shape[2]
    p_ref[...] = _halves(h_ref[...] * _dinv_from_parts(degp_ref[...]), dh)


def _hidden_body(a_ref, p1_ref, degp_ref, b1_ref, p2_ref):
    dh = p1_ref.shape[2]
    dinv = _dinv_from_parts(degp_ref[...])
    a = a_ref[...]
    p1 = p1_ref[...]
    agg_plus_p = jnp.concatenate(
        [a[0, 0] + a[0, 1] + p1[0], a[1, 0] + a[1, 1] + p1[1]], axis=1)
    pre = agg_plus_p * dinv + b1_ref[...]
    p2_ref[...] = _halves(jnp.maximum(pre, 0.0) * dinv, dh)


def _final_body(a_ref, p2_ref, degp_ref, wmu_ref, bmu_ref, wlv_ref, blv_ref,
                mu_ref, lv_ref):
    dinv = _dinv_from_parts(degp_ref[...])
    a = a_ref[...]
    p2 = p2_ref[...]
    z = jnp.concatenate(
        [a[0, 0] + a[0, 1] + p2[0], a[1, 0] + a[1, 1] + p2[1]], axis=1) * dinv
    mu_ref[...] = jnp.dot(z, wmu_ref[...], preferred_element_type=jnp.float32,
                          precision=_HIGHEST) + bmu_ref[...]
    lv_ref[...] = jnp.dot(z, wlv_ref[...], preferred_element_type=jnp.float32,
                          precision=_HIGHEST) + blv_ref[...]


def _f32(*shape):
    return jax.ShapeDtypeStruct(shape, jnp.float32)


def _row_spec(shape, row_dim):
    block = tuple(BLK if d == row_dim else s for d, s in enumerate(shape))

    def index_map(i):
        return tuple(i if d == row_dim else 0 for d in range(len(shape)))

    return pl.BlockSpec(block, index_map)


def _full_spec(shape):
    return pl.BlockSpec(shape, lambda i: (0,) * len(shape))


def kernel(x, edge_index, W1, b1, W_mu, b_mu, W_lv, b_lv):
    n, d_in = x.shape
    h1_dim = W1.shape[1]
    h2_dim = W_mu.shape[1]
    dh = h1_dim // 2
    e = edge_index.shape[1]

    chunk = NW * WIN * IDXC
    e_pad = -(-e // chunk) * chunk
    pad = e_pad - e
    k_win = e_pad // (NW * WIN)
    n_acc = n + PAD_ROWS

    src = edge_index[0]
    dst = edge_index[1]
    pad_src = jnp.zeros((pad,), jnp.int32)
    pad_dst = n + (jnp.arange(pad, dtype=jnp.int32) % PAD_ROWS)
    src3 = jnp.concatenate([src, pad_src]).reshape(k_win * WIN, NW).T \
        .reshape(NW, k_win, WIN)
    dst3 = jnp.concatenate([dst, pad_dst]).reshape(k_win * WIN, NW).T \
        .reshape(NW, k_win, WIN)

    ones_rows = jnp.ones((WIN, 16), jnp.float32)
    zeros16 = jnp.zeros((n_acc, 16), jnp.float32)
    zeros_d = jnp.zeros((n_acc, dh), jnp.float32)

    grid = (n // BLK,)
    a_shape = (2, NC, n_acc, dh)
    a_spec = _row_spec(a_shape, 2)
    degp_shape = (NC, n_acc, 16)
    degp_spec = _row_spec(degp_shape, 1)
    ph_spec = _row_spec((2, n, dh), 1)

    degp = _deg_partials(dst3, ones_rows, zeros16, n_acc)
    h1 = pl.pallas_call(
        _mm_body, out_shape=_f32(n, h1_dim), grid=grid,
        in_specs=[_row_spec((n, d_in), 0), _full_spec((d_in, h1_dim))],
        out_specs=_row_spec((n, h1_dim), 0))(x, W1)

    p1h = pl.pallas_call(
        _scale_body, out_shape=_f32(2, n, dh), grid=grid,
        in_specs=[_row_spec((n, h1_dim), 0), degp_spec],
        out_specs=ph_spec)(h1, degp)

    a1 = _agg_partials(src3, dst3, p1h, zeros_d, n_acc)

    p2h = pl.pallas_call(
        _hidden_body, out_shape=_f32(2, n, dh), grid=grid,
        in_specs=[a_spec, ph_spec, degp_spec, _full_spec((1, h1_dim))],
        out_specs=ph_spec)(a1, p1h, degp, b1.reshape(1, h1_dim))

    a2 = _agg_partials(src3, dst3, p2h, zeros_d, n_acc)

    out_spec = _row_spec((n, h2_dim), 0)
    mu, lv = pl.pallas_call(
        _final_body, out_shape=(_f32(n, h2_dim), _f32(n, h2_dim)), grid=grid,
        in_specs=[a_spec, ph_spec, degp_spec,
                  _full_spec((d_in, h2_dim)), _full_spec((1, h2_dim)),
                  _full_spec((d_in, h2_dim)), _full_spec((1, h2_dim))],
        out_specs=(out_spec, out_spec))(
        a2, p2h, degp, W_mu, b_mu.reshape(1, h2_dim), W_lv,
        b_lv.reshape(1, h2_dim))
    return (mu, lv)


# speedup vs baseline: 1.9062x; 1.3102x over previous
import functools

import jax
import jax.numpy as jnp
from jax import lax
from jax.experimental import pallas as pl
from jax.experimental.pallas import tpu as pltpu
from jax.experimental.pallas import tpu_sc as plsc

NC = 2
NS = 16
NW = NC * NS
WIN = 128
IDXC = 16
PAD_ROWS = 112
_HIGHEST = jax.lax.Precision.HIGHEST
_UNTILED = pltpu.CompilerParams(use_tc_tiling_on_sc=False)

_MESH = plsc.VectorSubcoreMesh(core_axis_name="c", subcore_axis_name="s")


def _flat_wid():
    return lax.axis_index("c") * NS + lax.axis_index("s")


def _deg_partials(dst3, ones_rows, zeros16, n_acc):
    k_win = dst3.shape[1]
    rows_sub = n_acc // NS

    @functools.partial(
        pl.kernel,
        mesh=_MESH,
        out_type=jax.ShapeDtypeStruct((NC, n_acc, 16), jnp.float32),
        scratch_types=[
            pltpu.VMEM((k_win, WIN), jnp.int32),
            pltpu.VMEM((WIN, 16), jnp.float32),
            pltpu.VMEM_SHARED((n_acc, 16), jnp.float32),
            pltpu.SemaphoreType.DMA,
        ],
        compiler_params=_UNTILED,
    )
    def deg_kernel(dst_hbm, ones_hbm, zeros_hbm, out_hbm, dst_v, ones_v,
                   acc_sh, sem):
        c = lax.axis_index("c")
        s = lax.axis_index("s")
        wid = _flat_wid()
        pltpu.sync_copy(zeros_hbm.at[pl.ds(s * rows_sub, rows_sub)],
                        acc_sh.at[pl.ds(s * rows_sub, rows_sub)])
        pltpu.sync_copy(dst_hbm.at[wid], dst_v)
        pltpu.sync_copy(ones_hbm, ones_v)
        plsc.subcore_barrier()

        @pl.loop(0, k_win)
        def _(j):
            pltpu.async_copy(ones_v, acc_sh.at[dst_v.at[j]], sem, add=True)

        @pl.loop(0, k_win)
        def _(j):
            pltpu.make_async_copy(ones_hbm, ones_v, sem).wait()

        plsc.subcore_barrier()
        pltpu.sync_copy(acc_sh.at[pl.ds(s * rows_sub, rows_sub)],
                        out_hbm.at[c, pl.ds(s * rows_sub, rows_sub)])

    return deg_kernel(dst3, ones_rows, zeros16)


def _agg_partials(src3, dst3, ph, zeros_d, n_acc):
    k_win = src3.shape[1]
    n_tab = ph.shape[1]
    dh = ph.shape[2]
    rows_sub = n_acc // NS
    tab_sub = n_tab // NS

    @functools.partial(
        pl.kernel,
        mesh=_MESH,
        out_type=jax.ShapeDtypeStruct((2, NC, n_acc, dh), jnp.float32),
        scratch_types=[
            pltpu.VMEM((k_win, WIN), jnp.int32),
            pltpu.VMEM((k_win, WIN), jnp.int32),
            pltpu.VMEM((WIN, dh), jnp.float32),
            pltpu.VMEM((WIN, dh), jnp.float32),
            pltpu.VMEM_SHARED((n_tab, dh), jnp.float32),
            pltpu.VMEM_SHARED((n_acc, dh), jnp.float32),
            pltpu.SemaphoreType.DMA,
            pltpu.SemaphoreType.DMA,
            pltpu.SemaphoreType.DMA,
        ],
        compiler_params=_UNTILED,
    )
    def agg_kernel(src_hbm, dst_hbm, ph_hbm, zeros_hbm, out_hbm,
                   src_v, dst_v, rows0_v, rows1_v, tab_sh, acc_sh,
                   semg, sems0, sems1):
        c = lax.axis_index("c")
        s = lax.axis_index("s")
        wid = _flat_wid()
        rows = (rows0_v, rows1_v)
        sems = (sems0, sems1)

        def gather(j, b):
            pltpu.async_copy(tab_sh.at[src_v.at[j]], rows[b], semg).wait()

        def scatter(j, b):
            pltpu.async_copy(rows[b], acc_sh.at[dst_v.at[j]], sems[b],
                             add=True)

        def drain(b):
            pltpu.make_async_copy(zeros_hbm.at[pl.ds(0, WIN)], rows[b],
                                  sems[b]).wait()

        @pl.loop(0, 2)
        def _(h):
            pltpu.sync_copy(ph_hbm.at[h, pl.ds(s * tab_sub, tab_sub)],
                            tab_sh.at[pl.ds(s * tab_sub, tab_sub)])
            pltpu.sync_copy(zeros_hbm.at[pl.ds(s * rows_sub, rows_sub)],
                            acc_sh.at[pl.ds(s * rows_sub, rows_sub)])
            pltpu.sync_copy(src_hbm.at[wid], src_v)
            pltpu.sync_copy(dst_hbm.at[wid], dst_v)
            plsc.subcore_barrier()

            for b in (0, 1):
                gather(b, b)
                scatter(b, b)

            @pl.loop(1, k_win // 2)
            def _(t):
                for b in (0, 1):
                    j = 2 * t + b
                    drain(b)
                    gather(j, b)
                    scatter(j, b)

            for b in (0, 1):
                drain(b)

            plsc.subcore_barrier()
            pltpu.sync_copy(acc_sh.at[pl.ds(s * rows_sub, rows_sub)],
                            out_hbm.at[h, c, pl.ds(s * rows_sub, rows_sub)])

    return agg_kernel(src3, dst3, ph, zeros_d)


BLK = 400


def _dinv_from_parts(degp):
    deg = degp[0, :, 0:1] + degp[1, :, 0:1] + 1.0
    return 1.0 / jnp.sqrt(deg)


def _halves(v, dh):
    return jnp.stack([v[:, :dh], v[:, dh:]], axis=0)


def _mm_body(x_ref, w_ref, o_ref):
    o_ref[...] = jnp.dot(x_ref[...], w_ref[...],
                         preferred_element_type=jnp.float32,
                         precision=_HIGHEST)


def _scale_body(h_ref, degp_ref, p_ref):
    dh = p_ref.shape[2]
    p_ref[...] = _halves(h_ref[...] * _dinv_from_parts(degp_ref[...]), dh)


def _hidden_body(a_ref, p1_ref, degp_ref, b1_ref, p2_ref):
    dh = p1_ref.shape[2]
    dinv = _dinv_from_parts(degp_ref[...])
    a = a_ref[...]
    p1 = p1_ref[...]
    agg_plus_p = jnp.concatenate(
        [a[0, 0] + a[0, 1] + p1[0], a[1, 0] + a[1, 1] + p1[1]], axis=1)
    pre = agg_plus_p * dinv + b1_ref[...]
    p2_ref[...] = _halves(jnp.maximum(pre, 0.0) * dinv, dh)


def _final_body(a_ref, p2_ref, degp_ref, wmu_ref, bmu_ref, wlv_ref, blv_ref,
                mu_ref, lv_ref):
    dinv = _dinv_from_parts(degp_ref[...])
    a = a_ref[...]
    p2 = p2_ref[...]
    z = jnp.concatenate(
        [a[0, 0] + a[0, 1] + p2[0], a[1, 0] + a[1, 1] + p2[1]], axis=1) * dinv
    mu_ref[...] = jnp.dot(z, wmu_ref[...], preferred_element_type=jnp.float32,
                          precision=_HIGHEST) + bmu_ref[...]
    lv_ref[...] = jnp.dot(z, wlv_ref[...], preferred_element_type=jnp.float32,
                          precision=_HIGHEST) + blv_ref[...]


def _f32(*shape):
    return jax.ShapeDtypeStruct(shape, jnp.float32)


def _row_spec(shape, row_dim):
    block = tuple(BLK if d == row_dim else s for d, s in enumerate(shape))

    def index_map(i):
        return tuple(i if d == row_dim else 0 for d in range(len(shape)))

    return pl.BlockSpec(block, index_map)


def _full_spec(shape):
    return pl.BlockSpec(shape, lambda i: (0,) * len(shape))


def kernel(x, edge_index, W1, b1, W_mu, b_mu, W_lv, b_lv):
    n, d_in = x.shape
    h1_dim = W1.shape[1]
    h2_dim = W_mu.shape[1]
    dh = h1_dim // 2
    e = edge_index.shape[1]

    chunk = NW * WIN * IDXC
    e_pad = -(-e // chunk) * chunk
    pad = e_pad - e
    k_win = e_pad // (NW * WIN)
    n_acc = n + PAD_ROWS

    src = edge_index[0]
    dst = edge_index[1]
    pad_src = jnp.zeros((pad,), jnp.int32)
    pad_dst = n + (jnp.arange(pad, dtype=jnp.int32) % PAD_ROWS)
    src3 = jnp.concatenate([src, pad_src]).reshape(k_win * WIN, NW).T \
        .reshape(NW, k_win, WIN)
    dst3 = jnp.concatenate([dst, pad_dst]).reshape(k_win * WIN, NW).T \
        .reshape(NW, k_win, WIN)

    ones_rows = jnp.ones((WIN, 16), jnp.float32)
    zeros16 = jnp.zeros((n_acc, 16), jnp.float32)
    zeros_d = jnp.zeros((n_acc, dh), jnp.float32)

    grid = (n // BLK,)
    a_shape = (2, NC, n_acc, dh)
    a_spec = _row_spec(a_shape, 2)
    degp_shape = (NC, n_acc, 16)
    degp_spec = _row_spec(degp_shape, 1)
    ph_spec = _row_spec((2, n, dh), 1)

    degp = _deg_partials(dst3, ones_rows, zeros16, n_acc)
    h1 = pl.pallas_call(
        _mm_body, out_shape=_f32(n, h1_dim), grid=grid,
        in_specs=[_row_spec((n, d_in), 0), _full_spec((d_in, h1_dim))],
        out_specs=_row_spec((n, h1_dim), 0))(x, W1)

    p1h = pl.pallas_call(
        _scale_body, out_shape=_f32(2, n, dh), grid=grid,
        in_specs=[_row_spec((n, h1_dim), 0), degp_spec],
        out_specs=ph_spec)(h1, degp)

    a1 = _agg_partials(src3, dst3, p1h, zeros_d, n_acc)

    p2h = pl.pallas_call(
        _hidden_body, out_shape=_f32(2, n, dh), grid=grid,
        in_specs=[a_spec, ph_spec, degp_spec, _full_spec((1, h1_dim))],
        out_specs=ph_spec)(a1, p1h, degp, b1.reshape(1, h1_dim))

    a2 = _agg_partials(src3, dst3, p2h, zeros_d, n_acc)

    out_spec = _row_spec((n, h2_dim), 0)
    mu, lv = pl.pallas_call(
        _final_body, out_shape=(_f32(n, h2_dim), _f32(n, h2_dim)), grid=grid,
        in_specs=[a_spec, ph_spec, degp_spec,
                  _full_spec((d_in, h2_dim)), _full_spec((1, h2_dim)),
                  _full_spec((d_in, h2_dim)), _full_spec((1, h2_dim))],
        out_specs=(out_spec, out_spec))(
        a2, p2h, degp, W_mu, b_mu.reshape(1, h2_dim), W_lv,
        b_lv.reshape(1, h2_dim))
    return (mu, lv)


# speedup vs baseline: 1.9841x; 1.0409x over previous
import functools

import jax
import jax.numpy as jnp
from jax import lax
from jax.experimental import pallas as pl
from jax.experimental.pallas import tpu as pltpu
from jax.experimental.pallas import tpu_sc as plsc

NC = 2
NS = 16
NW = NC * NS
WIN = 128

PAD_ROWS = 112
_HIGHEST = jax.lax.Precision.HIGHEST
_UNTILED = pltpu.CompilerParams(use_tc_tiling_on_sc=False)

_MESH = plsc.VectorSubcoreMesh(core_axis_name="c", subcore_axis_name="s")


def _flat_wid():
    return lax.axis_index("c") * NS + lax.axis_index("s")


def _win_split(e_win):
    base = e_win // NW
    rem = e_win % NW
    ku = base + (1 if rem else 0)
    ku_even = ku + (ku % 2)
    return base, rem, ku, ku_even


def _worker_range(base, rem, wid):
    count = base + jnp.where(wid < rem, 1, 0)
    start = wid * base + lax.min(wid, rem)
    return start, count


def _stage_idx(idx_hbm, idx_v, start, count, base, rem):
    if rem == 0:
        pltpu.sync_copy(idx_hbm.at[pl.ds(start, base)],
                        idx_v.at[pl.ds(0, base)])
    else:
        @pl.when(count == base + 1)
        def _():
            pltpu.sync_copy(idx_hbm.at[pl.ds(start, base + 1)],
                            idx_v.at[pl.ds(0, base + 1)])

        @pl.when(count == base)
        def _():
            pltpu.sync_copy(idx_hbm.at[pl.ds(start, base)],
                            idx_v.at[pl.ds(0, base)])


def _deg_partials(dst1, ones_rows, zeros16, n_acc):
    e_win = dst1.shape[0]
    base, rem, ku, _ = _win_split(e_win)
    rows_sub = n_acc // NS

    @functools.partial(
        pl.kernel,
        mesh=_MESH,
        out_type=jax.ShapeDtypeStruct((NC, n_acc, 16), jnp.float32),
        scratch_types=[
            pltpu.VMEM((ku, WIN), jnp.int32),
            pltpu.VMEM((WIN, 16), jnp.float32),
            pltpu.VMEM_SHARED((n_acc, 16), jnp.float32),
            pltpu.SemaphoreType.DMA,
        ],
        compiler_params=_UNTILED,
    )
    def deg_kernel(dst_hbm, ones_hbm, zeros_hbm, out_hbm, dst_v, ones_v,
                   acc_sh, sem):
        c = lax.axis_index("c")
        s = lax.axis_index("s")
        wid = _flat_wid()
        start, count = _worker_range(base, rem, wid)
        pltpu.sync_copy(zeros_hbm.at[pl.ds(s * rows_sub, rows_sub)],
                        acc_sh.at[pl.ds(s * rows_sub, rows_sub)])
        _stage_idx(dst_hbm, dst_v, start, count, base, rem)
        pltpu.sync_copy(ones_hbm, ones_v)
        plsc.subcore_barrier()

        @pl.loop(0, ku)
        def _(j):
            @pl.when(j < count)
            def _():
                pltpu.async_copy(ones_v, acc_sh.at[dst_v.at[j]], sem,
                                 add=True)

        @pl.loop(0, ku)
        def _(j):
            @pl.when(j < count)
            def _():
                pltpu.make_async_copy(ones_hbm, ones_v, sem).wait()

        plsc.subcore_barrier()
        pltpu.sync_copy(acc_sh.at[pl.ds(s * rows_sub, rows_sub)],
                        out_hbm.at[c, pl.ds(s * rows_sub, rows_sub)])

    return deg_kernel(dst1, ones_rows, zeros16)


def _agg_partials(src2, dst2, ph, zeros_d, n_acc):
    e_win = src2.shape[0]
    base, rem, ku, ku_even = _win_split(e_win)
    n_tab = ph.shape[1]
    dh = ph.shape[2]
    rows_sub = n_acc // NS
    tab_sub = n_tab // NS

    @functools.partial(
        pl.kernel,
        mesh=_MESH,
        out_type=jax.ShapeDtypeStruct((2, NC, n_acc, dh), jnp.float32),
        scratch_types=[
            pltpu.VMEM((ku, WIN), jnp.int32),
            pltpu.VMEM((ku, WIN), jnp.int32),
            pltpu.VMEM((WIN, dh), jnp.float32),
            pltpu.VMEM((WIN, dh), jnp.float32),
            pltpu.VMEM_SHARED((n_tab, dh), jnp.float32),
            pltpu.VMEM_SHARED((n_acc, dh), jnp.float32),
            pltpu.SemaphoreType.DMA,
            pltpu.SemaphoreType.DMA,
            pltpu.SemaphoreType.DMA,
        ],
        compiler_params=_UNTILED,
    )
    def agg_kernel(src_hbm, dst_hbm, ph_hbm, zeros_hbm, out_hbm,
                   src_v, dst_v, rows0_v, rows1_v, tab_sh, acc_sh,
                   semg, sems0, sems1):
        c = lax.axis_index("c")
        s = lax.axis_index("s")
        wid = _flat_wid()
        start, count = _worker_range(base, rem, wid)
        rows = (rows0_v, rows1_v)
        sems = (sems0, sems1)

        def gather(j, b):
            pltpu.async_copy(tab_sh.at[src_v.at[j]], rows[b], semg).wait()

        def scatter(j, b):
            pltpu.async_copy(rows[b], acc_sh.at[dst_v.at[j]], sems[b],
                             add=True)

        def drain(b):
            pltpu.make_async_copy(zeros_hbm.at[pl.ds(0, WIN)], rows[b],
                                  sems[b]).wait()

        @pl.loop(0, 2)
        def _(h):
            pltpu.sync_copy(ph_hbm.at[h, pl.ds(s * tab_sub, tab_sub)],
                            tab_sh.at[pl.ds(s * tab_sub, tab_sub)])
            pltpu.sync_copy(zeros_hbm.at[pl.ds(s * rows_sub, rows_sub)],
                            acc_sh.at[pl.ds(s * rows_sub, rows_sub)])
            _stage_idx(src_hbm, src_v, start, count, base, rem)
            _stage_idx(dst_hbm, dst_v, start, count, base, rem)
            plsc.subcore_barrier()

            for b in (0, 1):
                gather(b, b)
                scatter(b, b)

            @pl.loop(1, ku_even // 2)
            def _(t):
                for b in (0, 1):
                    j = 2 * t + b

                    @pl.when(j < count)
                    def _():
                        drain(b)
                        gather(j, b)
                        scatter(j, b)

            for b in (0, 1):
                drain(b)

            plsc.subcore_barrier()
            pltpu.sync_copy(acc_sh.at[pl.ds(s * rows_sub, rows_sub)],
                            out_hbm.at[h, c, pl.ds(s * rows_sub, rows_sub)])

    return agg_kernel(src2, dst2, ph, zeros_d)


BLK = 400


def _dinv_from_parts(degp):
    deg = degp[0, :, 0:1] + degp[1, :, 0:1] + 1.0
    return 1.0 / jnp.sqrt(deg)


def _halves(v, dh):
    return jnp.stack([v[:, :dh], v[:, dh:]], axis=0)


def _mm_body(x_ref, w_ref, o_ref):
    o_ref[...] = jnp.dot(x_ref[...], w_ref[...],
                         preferred_element_type=jnp.float32,
                         precision=_HIGHEST)


def _scale_body(h_ref, degp_ref, p_ref):
    dh = p_ref.shape[2]
    p_ref[...] = _halves(h_ref[...] * _dinv_from_parts(degp_ref[...]), dh)


def _hidden_body(a_ref, p1_ref, degp_ref, b1_ref, p2_ref):
    dh = p1_ref.shape[2]
    dinv = _dinv_from_parts(degp_ref[...])
    a = a_ref[...]
    p1 = p1_ref[...]
    agg_plus_p = jnp.concatenate(
        [a[0, 0] + a[0, 1] + p1[0], a[1, 0] + a[1, 1] + p1[1]], axis=1)
    pre = agg_plus_p * dinv + b1_ref[...]
    p2_ref[...] = _halves(jnp.maximum(pre, 0.0) * dinv, dh)


def _final_body(a_ref, p2_ref, degp_ref, wmu_ref, bmu_ref, wlv_ref, blv_ref,
                mu_ref, lv_ref):
    dinv = _dinv_from_parts(degp_ref[...])
    a = a_ref[...]
    p2 = p2_ref[...]
    z = jnp.concatenate(
        [a[0, 0] + a[0, 1] + p2[0], a[1, 0] + a[1, 1] + p2[1]], axis=1) * dinv
    mu_ref[...] = jnp.dot(z, wmu_ref[...], preferred_element_type=jnp.float32,
                          precision=_HIGHEST) + bmu_ref[...]
    lv_ref[...] = jnp.dot(z, wlv_ref[...], preferred_element_type=jnp.float32,
                          precision=_HIGHEST) + blv_ref[...]


def _f32(*shape):
    return jax.ShapeDtypeStruct(shape, jnp.float32)


def _row_spec(shape, row_dim):
    block = tuple(BLK if d == row_dim else s for d, s in enumerate(shape))

    def index_map(i):
        return tuple(i if d == row_dim else 0 for d in range(len(shape)))

    return pl.BlockSpec(block, index_map)


def _full_spec(shape):
    return pl.BlockSpec(shape, lambda i: (0,) * len(shape))


def kernel(x, edge_index, W1, b1, W_mu, b_mu, W_lv, b_lv):
    n, d_in = x.shape
    h1_dim = W1.shape[1]
    h2_dim = W_mu.shape[1]
    dh = h1_dim // 2
    e = edge_index.shape[1]

    assert e % WIN == 0, "edge count must be a multiple of the window size"
    e_win = e // WIN
    n_acc = n + PAD_ROWS

    src2 = edge_index[0].reshape(e_win, WIN)
    dst2 = edge_index[1].reshape(e_win, WIN)

    ones_rows = jnp.ones((WIN, 16), jnp.float32)
    zeros16 = jnp.zeros((n_acc, 16), jnp.float32)
    zeros_d = jnp.zeros((n_acc, dh), jnp.float32)

    grid = (n // BLK,)
    a_shape = (2, NC, n_acc, dh)
    a_spec = _row_spec(a_shape, 2)
    degp_shape = (NC, n_acc, 16)
    degp_spec = _row_spec(degp_shape, 1)
    ph_spec = _row_spec((2, n, dh), 1)

    degp = _deg_partials(dst2, ones_rows, zeros16, n_acc)
    h1 = pl.pallas_call(
        _mm_body, out_shape=_f32(n, h1_dim), grid=grid,
        in_specs=[_row_spec((n, d_in), 0), _full_spec((d_in, h1_dim))],
        out_specs=_row_spec((n, h1_dim), 0))(x, W1)

    p1h = pl.pallas_call(
        _scale_body, out_shape=_f32(2, n, dh), grid=grid,
        in_specs=[_row_spec((n, h1_dim), 0), degp_spec],
        out_specs=ph_spec)(h1, degp)

    a1 = _agg_partials(src2, dst2, p1h, zeros_d, n_acc)

    p2h = pl.pallas_call(
        _hidden_body, out_shape=_f32(2, n, dh), grid=grid,
        in_specs=[a_spec, ph_spec, degp_spec, _full_spec((1, h1_dim))],
        out_specs=ph_spec)(a1, p1h, degp, b1.reshape(1, h1_dim))

    a2 = _agg_partials(src2, dst2, p2h, zeros_d, n_acc)

    out_spec = _row_spec((n, h2_dim), 0)
    mu, lv = pl.pallas_call(
        _final_body, out_shape=(_f32(n, h2_dim), _f32(n, h2_dim)), grid=grid,
        in_specs=[a_spec, ph_spec, degp_spec,
                  _full_spec((d_in, h2_dim)), _full_spec((1, h2_dim)),
                  _full_spec((d_in, h2_dim)), _full_spec((1, h2_dim))],
        out_specs=(out_spec, out_spec))(
        a2, p2h, degp, W_mu, b_mu.reshape(1, h2_dim), W_lv,
        b_lv.reshape(1, h2_dim))
    return (mu, lv)


# speedup vs baseline: 2.1115x; 1.0642x over previous
import functools

import jax
import jax.numpy as jnp
from jax import lax
from jax.experimental import pallas as pl
from jax.experimental.pallas import tpu as pltpu
from jax.experimental.pallas import tpu_sc as plsc

NC = 2
NS = 16
NW = NC * NS
WIN = 128

PAD_ROWS = 112
_HIGHEST = jax.lax.Precision.HIGHEST
_UNTILED = pltpu.CompilerParams(use_tc_tiling_on_sc=False)

_MESH = plsc.VectorSubcoreMesh(core_axis_name="c", subcore_axis_name="s")


def _flat_wid():
    return lax.axis_index("c") * NS + lax.axis_index("s")


def _win_split(e_win):
    base = e_win // NW
    rem = e_win % NW
    ku = base + (1 if rem else 0)
    ku_even = ku + (ku % 2)
    return base, rem, ku, ku_even


def _worker_range(base, rem, wid):
    count = base + jnp.where(wid < rem, 1, 0)
    start = wid * base + lax.min(wid, rem)
    return start, count


def _stage_idx(idx_hbm, idx_v, start, count, base, rem):
    if rem == 0:
        pltpu.sync_copy(idx_hbm.at[pl.ds(start, base)],
                        idx_v.at[pl.ds(0, base)])
    else:
        @pl.when(count == base + 1)
        def _():
            pltpu.sync_copy(idx_hbm.at[pl.ds(start, base + 1)],
                            idx_v.at[pl.ds(0, base + 1)])

        @pl.when(count == base)
        def _():
            pltpu.sync_copy(idx_hbm.at[pl.ds(start, base)],
                            idx_v.at[pl.ds(0, base)])


def _deg_partials(dst1, ones_rows, zeros16, n_acc):
    e_win = dst1.shape[0]
    base, rem, ku, _ = _win_split(e_win)
    rows_sub = n_acc // NS

    @functools.partial(
        pl.kernel,
        mesh=_MESH,
        out_type=jax.ShapeDtypeStruct((NC, n_acc, 16), jnp.float32),
        scratch_types=[
            pltpu.VMEM((ku, WIN), jnp.int32),
            pltpu.VMEM((WIN, 16), jnp.float32),
            pltpu.VMEM_SHARED((n_acc, 16), jnp.float32),
            pltpu.SemaphoreType.DMA,
        ],
        compiler_params=_UNTILED,
    )
    def deg_kernel(dst_hbm, ones_hbm, zeros_hbm, out_hbm, dst_v, ones_v,
                   acc_sh, sem):
        c = lax.axis_index("c")
        s = lax.axis_index("s")
        wid = _flat_wid()
        start, count = _worker_range(base, rem, wid)
        pltpu.sync_copy(zeros_hbm.at[pl.ds(s * rows_sub, rows_sub)],
                        acc_sh.at[pl.ds(s * rows_sub, rows_sub)])
        _stage_idx(dst_hbm, dst_v, start, count, base, rem)
        pltpu.sync_copy(ones_hbm, ones_v)
        plsc.subcore_barrier()

        @pl.loop(0, ku)
        def _(j):
            @pl.when(j < count)
            def _():
                pltpu.async_copy(ones_v, acc_sh.at[dst_v.at[j]], sem,
                                 add=True)

        @pl.loop(0, ku)
        def _(j):
            @pl.when(j < count)
            def _():
                pltpu.make_async_copy(ones_hbm, ones_v, sem).wait()

        plsc.subcore_barrier()
        pltpu.sync_copy(acc_sh.at[pl.ds(s * rows_sub, rows_sub)],
                        out_hbm.at[c, pl.ds(s * rows_sub, rows_sub)])

    return deg_kernel(dst1, ones_rows, zeros16)


def _agg_partials(src2, dst2, ph, zeros_d, n_acc):
    e_win = src2.shape[0]
    base, rem, ku, ku_even = _win_split(e_win)
    n_tab = ph.shape[1]
    dh = ph.shape[2]
    rows_sub = n_acc // NS
    tab_sub = n_tab // NS

    del ku_even
    ku4 = -(-ku // 4) * 4
    segn = 2
    seg_len = ku4 // segn
    n_t = seg_len // 4
    pad_lo = base - (segn - 1) * seg_len
    assert seg_len % 4 == 0 and base >= (segn - 1) * seg_len and pad_lo >= 2

    @functools.partial(
        pl.kernel,
        mesh=_MESH,
        out_type=jax.ShapeDtypeStruct((2, NC, n_acc, dh), jnp.float32),
        scratch_types=[
            pltpu.VMEM((seg_len, WIN), jnp.int32),
            pltpu.VMEM((seg_len, WIN), jnp.int32),
            pltpu.VMEM((WIN, dh), jnp.float32),
            pltpu.VMEM((WIN, dh), jnp.float32),
            pltpu.VMEM((WIN, dh), jnp.float32),
            pltpu.VMEM((WIN, dh), jnp.float32),
            pltpu.VMEM_SHARED((n_tab, dh), jnp.float32),
            pltpu.VMEM_SHARED((n_acc, dh), jnp.float32),
            pltpu.SemaphoreType.DMA,
            pltpu.SemaphoreType.DMA,
            pltpu.SemaphoreType.DMA,
            pltpu.SemaphoreType.DMA,
            pltpu.SemaphoreType.DMA,
            pltpu.SemaphoreType.DMA,
            pltpu.SemaphoreType.DMA,
            pltpu.SemaphoreType.DMA,
        ],
        compiler_params=_UNTILED,
    )
    def agg_kernel(src_hbm, dst_hbm, ph_hbm, zeros_hbm, out_hbm,
                   src_v, dst_v, r0, r1, r2, r3, tab_sh, acc_sh,
                   g0, g1, g2, g3, s0, s1, s2, s3):
        c = lax.axis_index("c")
        s = lax.axis_index("s")
        wid = _flat_wid()
        start, count = _worker_range(base, rem, wid)
        rows = (r0, r1, r2, r3)
        semg = (g0, g1, g2, g3)
        sems = (s0, s1, s2, s3)

        def fire_gather(j, b):
            pltpu.async_copy(tab_sh.at[src_v.at[j]], rows[b], semg[b])

        def fire_scatter(j, b):
            pltpu.async_copy(rows[b], acc_sh.at[dst_v.at[j]], sems[b],
                             add=True)

        def drain(sem, b):
            pltpu.make_async_copy(zeros_hbm.at[pl.ds(0, WIN)], rows[b],
                                  sem[b]).wait()

        def stage_segment(seg):
            s0 = start + seg * seg_len
            if seg < segn - 1:
                pltpu.sync_copy(src_hbm.at[pl.ds(s0, seg_len)], src_v)
                pltpu.sync_copy(dst_hbm.at[pl.ds(s0, seg_len)], dst_v)
                return
            local = count - seg * seg_len

            def copy_rows(k):
                pltpu.sync_copy(src_hbm.at[pl.ds(s0, k)],
                                src_v.at[pl.ds(0, k)])
                pltpu.sync_copy(dst_hbm.at[pl.ds(s0, k)],
                                dst_v.at[pl.ds(0, k)])

            if rem == 0:
                copy_rows(pad_lo)
            else:
                @pl.when(local == pad_lo + 1)
                def _():
                    copy_rows(pad_lo + 1)

                @pl.when(local == pad_lo)
                def _():
                    copy_rows(pad_lo)

            zeros16i = jnp.zeros((16,), jnp.int32)
            pad16i = n_tab + lax.iota(jnp.int32, 16)
            for j in range(pad_lo, seg_len):
                @pl.when(j >= local)
                def _():
                    for q in range(WIN // 16):
                        src_v[j, pl.ds(16 * q, 16)] = zeros16i
                        dst_v[j, pl.ds(16 * q, 16)] = pad16i

        def run_pipeline():
            fire_gather(0, 0)
            fire_gather(1, 1)

            @pl.loop(0, n_t)
            def _(t):
                for b in range(4):
                    j = 4 * t + b
                    b2 = (b + 2) % 4
                    if b < 2:
                        @pl.when(t > 0)
                        def _():
                            drain(sems, b2)

                        fire_gather(j + 2, b2)
                    else:
                        drain(sems, b2)

                        @pl.when(t < n_t - 1)
                        def _():
                            fire_gather(j + 2, b2)

                    drain(semg, b)
                    fire_scatter(j, b)

            drain(sems, 2)
            drain(sems, 3)

        @pl.loop(0, 2)
        def _(h):
            pltpu.sync_copy(ph_hbm.at[h, pl.ds(s * tab_sub, tab_sub)],
                            tab_sh.at[pl.ds(s * tab_sub, tab_sub)])
            pltpu.sync_copy(zeros_hbm.at[pl.ds(s * rows_sub, rows_sub)],
                            acc_sh.at[pl.ds(s * rows_sub, rows_sub)])
            plsc.subcore_barrier()

            for seg in range(segn):
                stage_segment(seg)
                run_pipeline()

            plsc.subcore_barrier()
            pltpu.sync_copy(acc_sh.at[pl.ds(s * rows_sub, rows_sub)],
                            out_hbm.at[h, c, pl.ds(s * rows_sub, rows_sub)])

    return agg_kernel(src2, dst2, ph, zeros_d)


BLK = 400


def _dinv_from_parts(degp):
    deg = degp[0, :, 0:1] + degp[1, :, 0:1] + 1.0
    return 1.0 / jnp.sqrt(deg)


def _halves(v, dh):
    return jnp.stack([v[:, :dh], v[:, dh:]], axis=0)


def _mm_body(x_ref, w_ref, o_ref):
    o_ref[...] = jnp.dot(x_ref[...], w_ref[...],
                         preferred_element_type=jnp.float32,
                         precision=_HIGHEST)


def _scale_body(h_ref, degp_ref, p_ref):
    dh = p_ref.shape[2]
    p_ref[...] = _halves(h_ref[...] * _dinv_from_parts(degp_ref[...]), dh)


def _hidden_body(a_ref, p1_ref, degp_ref, b1_ref, p2_ref):
    dh = p1_ref.shape[2]
    dinv = _dinv_from_parts(degp_ref[...])
    a = a_ref[...]
    p1 = p1_ref[...]
    agg_plus_p = jnp.concatenate(
        [a[0, 0] + a[0, 1] + p1[0], a[1, 0] + a[1, 1] + p1[1]], axis=1)
    pre = agg_plus_p * dinv + b1_ref[...]
    p2_ref[...] = _halves(jnp.maximum(pre, 0.0) * dinv, dh)


def _final_body(a_ref, p2_ref, degp_ref, wmu_ref, bmu_ref, wlv_ref, blv_ref,
                mu_ref, lv_ref):
    dinv = _dinv_from_parts(degp_ref[...])
    a = a_ref[...]
    p2 = p2_ref[...]
    z = jnp.concatenate(
        [a[0, 0] + a[0, 1] + p2[0], a[1, 0] + a[1, 1] + p2[1]], axis=1) * dinv
    mu_ref[...] = jnp.dot(z, wmu_ref[...], preferred_element_type=jnp.float32,
                          precision=_HIGHEST) + bmu_ref[...]
    lv_ref[...] = jnp.dot(z, wlv_ref[...], preferred_element_type=jnp.float32,
                          precision=_HIGHEST) + blv_ref[...]


def _f32(*shape):
    return jax.ShapeDtypeStruct(shape, jnp.float32)


def _row_spec(shape, row_dim):
    block = tuple(BLK if d == row_dim else s for d, s in enumerate(shape))

    def index_map(i):
        return tuple(i if d == row_dim else 0 for d in range(len(shape)))

    return pl.BlockSpec(block, index_map)


def _full_spec(shape):
    return pl.BlockSpec(shape, lambda i: (0,) * len(shape))


def kernel(x, edge_index, W1, b1, W_mu, b_mu, W_lv, b_lv):
    n, d_in = x.shape
    h1_dim = W1.shape[1]
    h2_dim = W_mu.shape[1]
    dh = h1_dim // 2
    e = edge_index.shape[1]

    assert e % WIN == 0, "edge count must be a multiple of the window size"
    e_win = e // WIN
    n_acc = n + PAD_ROWS

    src2 = edge_index[0].reshape(e_win, WIN)
    dst2 = edge_index[1].reshape(e_win, WIN)

    ones_rows = jnp.ones((WIN, 16), jnp.float32)
    zeros16 = jnp.zeros((n_acc, 16), jnp.float32)
    zeros_d = jnp.zeros((n_acc, dh), jnp.float32)

    grid = (n // BLK,)
    a_shape = (2, NC, n_acc, dh)
    a_spec = _row_spec(a_shape, 2)
    degp_shape = (NC, n_acc, 16)
    degp_spec = _row_spec(degp_shape, 1)
    ph_spec = _row_spec((2, n, dh), 1)

    degp = _deg_partials(dst2, ones_rows, zeros16, n_acc)
    h1 = pl.pallas_call(
        _mm_body, out_shape=_f32(n, h1_dim), grid=grid,
        in_specs=[_row_spec((n, d_in), 0), _full_spec((d_in, h1_dim))],
        out_specs=_row_spec((n, h1_dim), 0))(x, W1)

    p1h = pl.pallas_call(
        _scale_body, out_shape=_f32(2, n, dh), grid=grid,
        in_specs=[_row_spec((n, h1_dim), 0), degp_spec],
        out_specs=ph_spec)(h1, degp)

    a1 = _agg_partials(src2, dst2, p1h, zeros_d, n_acc)

    p2h = pl.pallas_call(
        _hidden_body, out_shape=_f32(2, n, dh), grid=grid,
        in_specs=[a_spec, ph_spec, degp_spec, _full_spec((1, h1_dim))],
        out_specs=ph_spec)(a1, p1h, degp, b1.reshape(1, h1_dim))

    a2 = _agg_partials(src2, dst2, p2h, zeros_d, n_acc)

    out_spec = _row_spec((n, h2_dim), 0)
    mu, lv = pl.pallas_call(
        _final_body, out_shape=(_f32(n, h2_dim), _f32(n, h2_dim)), grid=grid,
        in_specs=[a_spec, ph_spec, degp_spec,
                  _full_spec((d_in, h2_dim)), _full_spec((1, h2_dim)),
                  _full_spec((d_in, h2_dim)), _full_spec((1, h2_dim))],
        out_specs=(out_spec, out_spec))(
        a2, p2h, degp, W_mu, b_mu.reshape(1, h2_dim), W_lv,
        b_lv.reshape(1, h2_dim))
    return (mu, lv)


# speedup vs baseline: 2.2804x; 1.0800x over previous
import functools

import jax
import jax.numpy as jnp
from jax import lax
from jax.experimental import pallas as pl
from jax.experimental.pallas import tpu as pltpu
from jax.experimental.pallas import tpu_sc as plsc

NC = 2
NS = 16
NW = NC * NS
WIN = 128

PAD_ROWS = 112
_HIGHEST = jax.lax.Precision.HIGHEST
_UNTILED = pltpu.CompilerParams(use_tc_tiling_on_sc=False)

_MESH = plsc.VectorSubcoreMesh(core_axis_name="c", subcore_axis_name="s")


def _flat_wid():
    return lax.axis_index("c") * NS + lax.axis_index("s")


def _win_split(e_win):
    base = e_win // NW
    rem = e_win % NW
    ku = base + (1 if rem else 0)
    ku_even = ku + (ku % 2)
    return base, rem, ku, ku_even


def _worker_range(base, rem, wid):
    count = base + jnp.where(wid < rem, 1, 0)
    start = wid * base + lax.min(wid, rem)
    return start, count


def _stage_idx(idx_hbm, idx_v, start, count, base, rem):
    if rem == 0:
        pltpu.sync_copy(idx_hbm.at[pl.ds(start, base)],
                        idx_v.at[pl.ds(0, base)])
    else:
        @pl.when(count == base + 1)
        def _():
            pltpu.sync_copy(idx_hbm.at[pl.ds(start, base + 1)],
                            idx_v.at[pl.ds(0, base + 1)])

        @pl.when(count == base)
        def _():
            pltpu.sync_copy(idx_hbm.at[pl.ds(start, base)],
                            idx_v.at[pl.ds(0, base)])


def _deg_partials(edge3, ones_rows, zeros16, n_acc):
    e_win = edge3.shape[1]
    base, rem, ku, _ = _win_split(e_win)
    rows_sub = n_acc // NS

    @functools.partial(
        pl.kernel,
        mesh=_MESH,
        out_type=jax.ShapeDtypeStruct((NC, n_acc, 16), jnp.float32),
        scratch_types=[
            pltpu.VMEM((ku, WIN), jnp.int32),
            pltpu.VMEM((WIN, 16), jnp.float32),
            pltpu.VMEM_SHARED((n_acc, 16), jnp.float32),
            pltpu.SemaphoreType.DMA,
        ],
        compiler_params=_UNTILED,
    )
    def deg_kernel(edge_hbm, ones_hbm, zeros_hbm, out_hbm, dst_v, ones_v,
                   acc_sh, sem):
        c = lax.axis_index("c")
        s = lax.axis_index("s")
        wid = _flat_wid()
        start, count = _worker_range(base, rem, wid)
        pltpu.sync_copy(zeros_hbm.at[pl.ds(s * rows_sub, rows_sub)],
                        acc_sh.at[pl.ds(s * rows_sub, rows_sub)])
        _stage_idx(edge_hbm.at[1], dst_v, start, count, base, rem)
        pltpu.sync_copy(ones_hbm, ones_v)
        plsc.subcore_barrier()

        @pl.loop(0, ku)
        def _(j):
            @pl.when(j < count)
            def _():
                pltpu.async_copy(ones_v, acc_sh.at[dst_v.at[j]], sem,
                                 add=True)

        @pl.loop(0, ku)
        def _(j):
            @pl.when(j < count)
            def _():
                pltpu.make_async_copy(ones_hbm, ones_v, sem).wait()

        plsc.subcore_barrier()
        pltpu.sync_copy(acc_sh.at[pl.ds(s * rows_sub, rows_sub)],
                        out_hbm.at[c, pl.ds(s * rows_sub, rows_sub)])

    return deg_kernel(edge3, ones_rows, zeros16)


def _agg_partials(edge3, ph, zeros_d, n_acc):
    e_win = edge3.shape[1]
    base, rem, ku, ku_even = _win_split(e_win)
    n_tab = ph.shape[1]
    dh = ph.shape[2]
    rows_sub = n_acc // NS
    tab_sub = n_tab // NS

    del ku_even
    ku4 = -(-ku // 4) * 4
    segn = 2
    seg_len = ku4 // segn
    n_t = seg_len // 4
    pad_lo = base - (segn - 1) * seg_len
    assert seg_len % 4 == 0 and base >= (segn - 1) * seg_len and pad_lo >= 2

    @functools.partial(
        pl.kernel,
        mesh=_MESH,
        out_type=jax.ShapeDtypeStruct((2, NC, n_acc, dh), jnp.float32),
        scratch_types=[
            pltpu.VMEM((seg_len, WIN), jnp.int32),
            pltpu.VMEM((seg_len, WIN), jnp.int32),
            pltpu.VMEM((WIN, dh), jnp.float32),
            pltpu.VMEM((WIN, dh), jnp.float32),
            pltpu.VMEM((WIN, dh), jnp.float32),
            pltpu.VMEM((WIN, dh), jnp.float32),
            pltpu.VMEM_SHARED((n_tab, dh), jnp.float32),
            pltpu.VMEM_SHARED((n_acc, dh), jnp.float32),
            pltpu.SemaphoreType.DMA,
            pltpu.SemaphoreType.DMA,
            pltpu.SemaphoreType.DMA,
            pltpu.SemaphoreType.DMA,
            pltpu.SemaphoreType.DMA,
            pltpu.SemaphoreType.DMA,
            pltpu.SemaphoreType.DMA,
            pltpu.SemaphoreType.DMA,
        ],
        compiler_params=_UNTILED,
    )
    def agg_kernel(edge_hbm, ph_hbm, zeros_hbm, out_hbm,
                   src_v, dst_v, r0, r1, r2, r3, tab_sh, acc_sh,
                   g0, g1, g2, g3, s0, s1, s2, s3):
        c = lax.axis_index("c")
        s = lax.axis_index("s")
        wid = _flat_wid()
        start, count = _worker_range(base, rem, wid)
        rows = (r0, r1, r2, r3)
        semg = (g0, g1, g2, g3)
        sems = (s0, s1, s2, s3)

        def fire_gather(j, b):
            pltpu.async_copy(tab_sh.at[src_v.at[j]], rows[b], semg[b])

        def fire_scatter(j, b):
            pltpu.async_copy(rows[b], acc_sh.at[dst_v.at[j]], sems[b],
                             add=True)

        def drain(sem, b):
            pltpu.make_async_copy(zeros_hbm.at[pl.ds(0, WIN)], rows[b],
                                  sem[b]).wait()

        def stage_segment(seg):
            s0 = start + seg * seg_len
            if seg < segn - 1:
                pltpu.sync_copy(edge_hbm.at[0, pl.ds(s0, seg_len)], src_v)
                pltpu.sync_copy(edge_hbm.at[1, pl.ds(s0, seg_len)], dst_v)
                return
            local = count - seg * seg_len

            def copy_rows(k):
                pltpu.sync_copy(edge_hbm.at[0, pl.ds(s0, k)],
                                src_v.at[pl.ds(0, k)])
                pltpu.sync_copy(edge_hbm.at[1, pl.ds(s0, k)],
                                dst_v.at[pl.ds(0, k)])

            if rem == 0:
                copy_rows(pad_lo)
            else:
                @pl.when(local == pad_lo + 1)
                def _():
                    copy_rows(pad_lo + 1)

                @pl.when(local == pad_lo)
                def _():
                    copy_rows(pad_lo)

            zeros16i = jnp.zeros((16,), jnp.int32)
            pad16i = n_tab + lax.iota(jnp.int32, 16)
            for j in range(pad_lo, seg_len):
                @pl.when(j >= local)
                def _():
                    for q in range(WIN // 16):
                        src_v[j, pl.ds(16 * q, 16)] = zeros16i
                        dst_v[j, pl.ds(16 * q, 16)] = pad16i

        def run_pipeline():
            fire_gather(0, 0)
            fire_gather(1, 1)

            @pl.loop(0, n_t)
            def _(t):
                for b in range(4):
                    j = 4 * t + b
                    b2 = (b + 2) % 4
                    if b < 2:
                        @pl.when(t > 0)
                        def _():
                            drain(sems, b2)

                        fire_gather(j + 2, b2)
                    else:
                        drain(sems, b2)

                        @pl.when(t < n_t - 1)
                        def _():
                            fire_gather(j + 2, b2)

                    drain(semg, b)
                    fire_scatter(j, b)

            drain(sems, 2)
            drain(sems, 3)

        @pl.loop(0, 2)
        def _(h):
            pltpu.sync_copy(ph_hbm.at[h, pl.ds(s * tab_sub, tab_sub)],
                            tab_sh.at[pl.ds(s * tab_sub, tab_sub)])
            pltpu.sync_copy(zeros_hbm.at[pl.ds(s * rows_sub, rows_sub)],
                            acc_sh.at[pl.ds(s * rows_sub, rows_sub)])
            plsc.subcore_barrier()

            for seg in range(segn):
                stage_segment(seg)
                run_pipeline()

            plsc.subcore_barrier()
            pltpu.sync_copy(acc_sh.at[pl.ds(s * rows_sub, rows_sub)],
                            out_hbm.at[h, c, pl.ds(s * rows_sub, rows_sub)])

    return agg_kernel(edge3, ph, zeros_d)


BLK = 1000


def _dinv_from_parts(degp):
    deg = degp[0, :, 0:1] + degp[1, :, 0:1] + 1.0
    return 1.0 / jnp.sqrt(deg)


def _halves(v, dh):
    return jnp.stack([v[:, :dh], v[:, dh:]], axis=0)


def _mm_body(x_ref, w_ref, o_ref):
    o_ref[...] = jnp.dot(x_ref[...], w_ref[...],
                         preferred_element_type=jnp.float32,
                         precision=_HIGHEST)


def _scale_body(h_ref, degp_ref, p_ref):
    dh = p_ref.shape[2]
    p_ref[...] = _halves(h_ref[...] * _dinv_from_parts(degp_ref[...]), dh)


def _hidden_body(a_ref, p1_ref, degp_ref, b1_ref, p2_ref):
    dh = p1_ref.shape[2]
    dinv = _dinv_from_parts(degp_ref[...])
    a = a_ref[...]
    p1 = p1_ref[...]
    agg_plus_p = jnp.concatenate(
        [a[0, 0] + a[0, 1] + p1[0], a[1, 0] + a[1, 1] + p1[1]], axis=1)
    pre = agg_plus_p * dinv + b1_ref[...]
    p2_ref[...] = _halves(jnp.maximum(pre, 0.0) * dinv, dh)


def _final_body(a_ref, p2_ref, degp_ref, wmu_ref, bmu_ref, wlv_ref, blv_ref,
                mu_ref, lv_ref):
    dinv = _dinv_from_parts(degp_ref[...])
    a = a_ref[...]
    p2 = p2_ref[...]
    z = jnp.concatenate(
        [a[0, 0] + a[0, 1] + p2[0], a[1, 0] + a[1, 1] + p2[1]], axis=1) * dinv
    mu_ref[...] = jnp.dot(z, wmu_ref[...], preferred_element_type=jnp.float32,
                          precision=_HIGHEST) + bmu_ref[...]
    lv_ref[...] = jnp.dot(z, wlv_ref[...], preferred_element_type=jnp.float32,
                          precision=_HIGHEST) + blv_ref[...]


def _f32(*shape):
    return jax.ShapeDtypeStruct(shape, jnp.float32)


def _row_spec(shape, row_dim):
    block = tuple(BLK if d == row_dim else s for d, s in enumerate(shape))

    def index_map(i):
        return tuple(i if d == row_dim else 0 for d in range(len(shape)))

    return pl.BlockSpec(block, index_map)


def _full_spec(shape):
    return pl.BlockSpec(shape, lambda i: (0,) * len(shape))


def kernel(x, edge_index, W1, b1, W_mu, b_mu, W_lv, b_lv):
    n, d_in = x.shape
    h1_dim = W1.shape[1]
    h2_dim = W_mu.shape[1]
    dh = h1_dim // 2
    e = edge_index.shape[1]

    assert e % WIN == 0, "edge count must be a multiple of the window size"
    e_win = e // WIN
    n_acc = n + PAD_ROWS

    edge3 = edge_index.reshape(2, e_win, WIN)

    ones_rows = jnp.ones((WIN, 16), jnp.float32)
    zeros16 = jnp.zeros((n_acc, 16), jnp.float32)
    zeros_d = jnp.zeros((n_acc, dh), jnp.float32)

    grid = (n // BLK,)
    a_shape = (2, NC, n_acc, dh)
    a_spec = _row_spec(a_shape, 2)
    degp_shape = (NC, n_acc, 16)
    degp_spec = _row_spec(degp_shape, 1)
    ph_spec = _row_spec((2, n, dh), 1)

    degp = _deg_partials(edge3, ones_rows, zeros16, n_acc)
    h1 = pl.pallas_call(
        _mm_body, out_shape=_f32(n, h1_dim), grid=grid,
        in_specs=[_row_spec((n, d_in), 0), _full_spec((d_in, h1_dim))],
        out_specs=_row_spec((n, h1_dim), 0))(x, W1)

    p1h = pl.pallas_call(
        _scale_body, out_shape=_f32(2, n, dh), grid=grid,
        in_specs=[_row_spec((n, h1_dim), 0), degp_spec],
        out_specs=ph_spec)(h1, degp)

    a1 = _agg_partials(edge3, p1h, zeros_d, n_acc)

    p2h = pl.pallas_call(
        _hidden_body, out_shape=_f32(2, n, dh), grid=grid,
        in_specs=[a_spec, ph_spec, degp_spec, _full_spec((1, h1_dim))],
        out_specs=ph_spec)(a1, p1h, degp, b1.reshape(1, h1_dim))

    a2 = _agg_partials(edge3, p2h, zeros_d, n_acc)

    out_spec = _row_spec((n, h2_dim), 0)
    mu, lv = pl.pallas_call(
        _final_body, out_shape=(_f32(n, h2_dim), _f32(n, h2_dim)), grid=grid,
        in_specs=[a_spec, ph_spec, degp_spec,
                  _full_spec((d_in, h2_dim)), _full_spec((1, h2_dim)),
                  _full_spec((d_in, h2_dim)), _full_spec((1, h2_dim))],
        out_specs=(out_spec, out_spec))(
        a2, p2h, degp, W_mu, b_mu.reshape(1, h2_dim), W_lv,
        b_lv.reshape(1, h2_dim))
    return (mu, lv)


# speedup vs baseline: 2.6603x; 1.1666x over previous
import functools

import jax
import jax.numpy as jnp
from jax import lax
from jax.experimental import pallas as pl
from jax.experimental.pallas import tpu as pltpu
from jax.experimental.pallas import tpu_sc as plsc

NC = 2
NS = 16
NW = NC * NS
WIN = 128

PAD_ROWS = 112
_HIGHEST = jax.lax.Precision.HIGHEST
_UNTILED = pltpu.CompilerParams(use_tc_tiling_on_sc=False)

_MESH = plsc.VectorSubcoreMesh(core_axis_name="c", subcore_axis_name="s")


def _flat_wid():
    return lax.axis_index("c") * NS + lax.axis_index("s")


def _win_split(e_win):
    base = e_win // NW
    rem = e_win % NW
    ku = base + (1 if rem else 0)
    ku_even = ku + (ku % 2)
    return base, rem, ku, ku_even


def _worker_range(base, rem, wid):
    count = base + jnp.where(wid < rem, 1, 0)
    start = wid * base + lax.min(wid, rem)
    return start, count


def _stage_idx(idx_hbm, idx_v, start, count, base, rem):
    if rem == 0:
        pltpu.sync_copy(idx_hbm.at[pl.ds(start, base)],
                        idx_v.at[pl.ds(0, base)])
    else:
        @pl.when(count == base + 1)
        def _():
            pltpu.sync_copy(idx_hbm.at[pl.ds(start, base + 1)],
                            idx_v.at[pl.ds(0, base + 1)])

        @pl.when(count == base)
        def _():
            pltpu.sync_copy(idx_hbm.at[pl.ds(start, base)],
                            idx_v.at[pl.ds(0, base)])


def _deg_partials(edge3, ones_rows, zeros16, n_acc):
    e_win = edge3.shape[1]
    base, rem, ku, _ = _win_split(e_win)
    rows_sub = n_acc // NS

    @functools.partial(
        pl.kernel,
        mesh=_MESH,
        out_type=jax.ShapeDtypeStruct((NC, n_acc, 16), jnp.float32),
        scratch_types=[
            pltpu.VMEM((ku, WIN), jnp.int32),
            pltpu.VMEM((WIN, 16), jnp.float32),
            pltpu.VMEM_SHARED((n_acc, 16), jnp.float32),
            pltpu.SemaphoreType.DMA,
        ],
        compiler_params=_UNTILED,
    )
    def deg_kernel(edge_hbm, ones_hbm, zeros_hbm, out_hbm, dst_v, ones_v,
                   acc_sh, sem):
        c = lax.axis_index("c")
        s = lax.axis_index("s")
        wid = _flat_wid()
        start, count = _worker_range(base, rem, wid)
        pltpu.sync_copy(zeros_hbm.at[pl.ds(s * rows_sub, rows_sub)],
                        acc_sh.at[pl.ds(s * rows_sub, rows_sub)])
        _stage_idx(edge_hbm.at[1], dst_v, start, count, base, rem)
        pltpu.sync_copy(ones_hbm, ones_v)
        plsc.subcore_barrier()

        @pl.loop(0, ku)
        def _(j):
            @pl.when(j < count)
            def _():
                pltpu.async_copy(ones_v, acc_sh.at[dst_v.at[j]], sem,
                                 add=True)

        @pl.loop(0, ku)
        def _(j):
            @pl.when(j < count)
            def _():
                pltpu.make_async_copy(ones_hbm, ones_v, sem).wait()

        plsc.subcore_barrier()
        pltpu.sync_copy(acc_sh.at[pl.ds(s * rows_sub, rows_sub)],
                        out_hbm.at[c, pl.ds(s * rows_sub, rows_sub)])

    return deg_kernel(edge3, ones_rows, zeros16)


def _agg_partials(edge3, p, zeros_d, n_acc):
    e_win = edge3.shape[1]
    base, rem, ku, ku_even = _win_split(e_win)
    n_tab = p.shape[0]
    dh = p.shape[1] // 2
    rows_sub = n_acc // NS
    tab_sub = n_tab // NS

    del ku_even
    ku4 = -(-ku // 4) * 4
    segn = 2
    seg_len = ku4 // segn
    n_t = seg_len // 4
    pad_lo = base - (segn - 1) * seg_len
    assert seg_len % 4 == 0 and base >= (segn - 1) * seg_len and pad_lo >= 2

    @functools.partial(
        pl.kernel,
        mesh=_MESH,
        out_type=jax.ShapeDtypeStruct((NC, n_acc, 2 * dh), jnp.float32),
        scratch_types=[
            pltpu.VMEM((seg_len, WIN), jnp.int32),
            pltpu.VMEM((seg_len, WIN), jnp.int32),
            pltpu.VMEM((WIN, dh), jnp.float32),
            pltpu.VMEM((WIN, dh), jnp.float32),
            pltpu.VMEM((WIN, dh), jnp.float32),
            pltpu.VMEM((WIN, dh), jnp.float32),
            pltpu.VMEM_SHARED((n_tab, dh), jnp.float32),
            pltpu.VMEM_SHARED((n_acc, dh), jnp.float32),
            pltpu.SemaphoreType.DMA,
            pltpu.SemaphoreType.DMA,
            pltpu.SemaphoreType.DMA,
            pltpu.SemaphoreType.DMA,
            pltpu.SemaphoreType.DMA,
            pltpu.SemaphoreType.DMA,
            pltpu.SemaphoreType.DMA,
            pltpu.SemaphoreType.DMA,
        ],
        compiler_params=_UNTILED,
    )
    def agg_kernel(edge_hbm, p_hbm, zeros_hbm, out_hbm,
                   src_v, dst_v, r0, r1, r2, r3, tab_sh, acc_sh,
                   g0, g1, g2, g3, s0, s1, s2, s3):
        c = lax.axis_index("c")
        s = lax.axis_index("s")
        wid = _flat_wid()
        start, count = _worker_range(base, rem, wid)
        rows = (r0, r1, r2, r3)
        semg = (g0, g1, g2, g3)
        sems = (s0, s1, s2, s3)

        def fire_gather(j, b):
            pltpu.async_copy(tab_sh.at[src_v.at[j]], rows[b], semg[b])

        def fire_scatter(j, b):
            pltpu.async_copy(rows[b], acc_sh.at[dst_v.at[j]], sems[b],
                             add=True)

        def drain(sem, b):
            pltpu.make_async_copy(zeros_hbm.at[pl.ds(0, WIN)], rows[b],
                                  sem[b]).wait()

        def stage_segment(seg):
            s0 = start + seg * seg_len
            if seg < segn - 1:
                pltpu.sync_copy(edge_hbm.at[0, pl.ds(s0, seg_len)], src_v)
                pltpu.sync_copy(edge_hbm.at[1, pl.ds(s0, seg_len)], dst_v)
                return
            local = count - seg * seg_len

            def copy_rows(k):
                pltpu.sync_copy(edge_hbm.at[0, pl.ds(s0, k)],
                                src_v.at[pl.ds(0, k)])
                pltpu.sync_copy(edge_hbm.at[1, pl.ds(s0, k)],
                                dst_v.at[pl.ds(0, k)])

            if rem == 0:
                copy_rows(pad_lo)
            else:
                @pl.when(local == pad_lo + 1)
                def _():
                    copy_rows(pad_lo + 1)

                @pl.when(local == pad_lo)
                def _():
                    copy_rows(pad_lo)

            zeros16i = jnp.zeros((16,), jnp.int32)
            pad16i = n_tab + lax.iota(jnp.int32, 16)
            for j in range(pad_lo, seg_len):
                @pl.when(j >= local)
                def _():
                    for q in range(WIN // 16):
                        src_v[j, pl.ds(16 * q, 16)] = zeros16i
                        dst_v[j, pl.ds(16 * q, 16)] = pad16i

        def run_pipeline():
            fire_gather(0, 0)
            fire_gather(1, 1)

            @pl.loop(0, n_t)
            def _(t):
                for b in range(4):
                    j = 4 * t + b
                    b2 = (b + 2) % 4
                    if b < 2:
                        @pl.when(t > 0)
                        def _():
                            drain(sems, b2)

                        fire_gather(j + 2, b2)
                    else:
                        drain(sems, b2)

                        @pl.when(t < n_t - 1)
                        def _():
                            fire_gather(j + 2, b2)

                    drain(semg, b)
                    fire_scatter(j, b)

            drain(sems, 2)
            drain(sems, 3)

        @pl.loop(0, 2)
        def _(h):
            pltpu.sync_copy(
                p_hbm.at[pl.ds(s * tab_sub, tab_sub), pl.ds(h * dh, dh)],
                tab_sh.at[pl.ds(s * tab_sub, tab_sub)])
            pltpu.sync_copy(zeros_hbm.at[pl.ds(s * rows_sub, rows_sub)],
                            acc_sh.at[pl.ds(s * rows_sub, rows_sub)])
            plsc.subcore_barrier()

            for seg in range(segn):
                stage_segment(seg)
                run_pipeline()

            plsc.subcore_barrier()
            pltpu.sync_copy(
                acc_sh.at[pl.ds(s * rows_sub, rows_sub)],
                out_hbm.at[c, pl.ds(s * rows_sub, rows_sub),
                           pl.ds(h * dh, dh)])

    return agg_kernel(edge3, p, zeros_d)


BLK = 1000


def _dinv_from_parts(degp):
    deg = degp[0, :, 0:1] + degp[1, :, 0:1] + 1.0
    return 1.0 / jnp.sqrt(deg)


def _mm_body(x_ref, w_ref, o_ref):
    o_ref[...] = jnp.dot(x_ref[...], w_ref[...],
                         preferred_element_type=jnp.float32,
                         precision=_HIGHEST)


def _scale_body(h_ref, degp_ref, p_ref):
    p_ref[...] = h_ref[...] * _dinv_from_parts(degp_ref[...])


def _hidden_body(a_ref, p1_ref, degp_ref, b1_ref, p2_ref):
    dinv = _dinv_from_parts(degp_ref[...])
    a = a_ref[...]
    pre = (a[0] + a[1] + p1_ref[...]) * dinv + b1_ref[...]
    p2_ref[...] = jnp.maximum(pre, 0.0) * dinv


def _final_body(a_ref, p2_ref, degp_ref, wmu_ref, bmu_ref, wlv_ref, blv_ref,
                mu_ref, lv_ref):
    dinv = _dinv_from_parts(degp_ref[...])
    a = a_ref[...]
    z = (a[0] + a[1] + p2_ref[...]) * dinv
    mu_ref[...] = jnp.dot(z, wmu_ref[...], preferred_element_type=jnp.float32,
                          precision=_HIGHEST) + bmu_ref[...]
    lv_ref[...] = jnp.dot(z, wlv_ref[...], preferred_element_type=jnp.float32,
                          precision=_HIGHEST) + blv_ref[...]


def _f32(*shape):
    return jax.ShapeDtypeStruct(shape, jnp.float32)


def _row_spec(shape, row_dim):
    block = tuple(BLK if d == row_dim else s for d, s in enumerate(shape))

    def index_map(i):
        return tuple(i if d == row_dim else 0 for d in range(len(shape)))

    return pl.BlockSpec(block, index_map)


def _full_spec(shape):
    return pl.BlockSpec(shape, lambda i: (0,) * len(shape))


def kernel(x, edge_index, W1, b1, W_mu, b_mu, W_lv, b_lv):
    n, d_in = x.shape
    h1_dim = W1.shape[1]
    h2_dim = W_mu.shape[1]
    e = edge_index.shape[1]

    assert e % WIN == 0, "edge count must be a multiple of the window size"
    e_win = e // WIN
    n_acc = n + PAD_ROWS

    edge3 = edge_index.reshape(2, e_win, WIN)

    ones_rows = jnp.ones((WIN, 16), jnp.float32)
    zeros16 = jnp.zeros((n_acc, 16), jnp.float32)
    zeros_d = jnp.zeros((n_acc, h1_dim // 2), jnp.float32)

    grid = (n // BLK,)
    a_spec = _row_spec((NC, n_acc, h1_dim), 1)
    degp_spec = _row_spec((NC, n_acc, 16), 1)
    p_spec = _row_spec((n, h1_dim), 0)

    degp = _deg_partials(edge3, ones_rows, zeros16, n_acc)
    h1 = pl.pallas_call(
        _mm_body, out_shape=_f32(n, h1_dim), grid=grid,
        in_specs=[_row_spec((n, d_in), 0), _full_spec((d_in, h1_dim))],
        out_specs=_row_spec((n, h1_dim), 0))(x, W1)

    p1 = pl.pallas_call(
        _scale_body, out_shape=_f32(n, h1_dim), grid=grid,
        in_specs=[p_spec, degp_spec], out_specs=p_spec)(h1, degp)

    a1 = _agg_partials(edge3, p1, zeros_d, n_acc)

    p2 = pl.pallas_call(
        _hidden_body, out_shape=_f32(n, h1_dim), grid=grid,
        in_specs=[a_spec, p_spec, degp_spec, _full_spec((1, h1_dim))],
        out_specs=p_spec)(a1, p1, degp, b1.reshape(1, h1_dim))

    a2 = _agg_partials(edge3, p2, zeros_d, n_acc)

    out_spec = _row_spec((n, h2_dim), 0)
    mu, lv = pl.pallas_call(
        _final_body, out_shape=(_f32(n, h2_dim), _f32(n, h2_dim)), grid=grid,
        in_specs=[a_spec, p_spec, degp_spec,
                  _full_spec((d_in, h2_dim)), _full_spec((1, h2_dim)),
                  _full_spec((d_in, h2_dim)), _full_spec((1, h2_dim))],
        out_specs=(out_spec, out_spec))(
        a2, p2, degp, W_mu, b_mu.reshape(1, h2_dim), W_lv,
        b_lv.reshape(1, h2_dim))
    return (mu, lv)
